# baseline (speedup 1.0000x reference)
import jax
import jax.numpy as jnp
from jax.experimental import pallas as pl
from jax.experimental.pallas import tpu as pltpu
from jax.experimental.pallas import tpu_sc as plsc

S = 2048
D = 1024
H = 16
DH = D // H
FF = 512
E = 8
K = 2
EPS = 1e-6

QB = 512
NQ = S // QB
BLK = 128
PBUF = S * K + E * BLK
NB = PBUF // BLK
SCW = 128
SCC = D // 4

_f32 = jnp.float32
_bf16 = jnp.bfloat16


def _k1_body(x_ref, g_ref, wq_ref, wk_ref, wv_ref, q_ref, k_ref, v_ref):
    x = x_ref[...]
    var = jnp.mean(x * x, axis=1, keepdims=True)
    xn = (x * jax.lax.rsqrt(var + EPS)) * g_ref[...]
    xb = xn.astype(_bf16)
    q_ref[...] = jnp.dot(xb, wq_ref[...],
                         preferred_element_type=_f32).astype(_bf16)
    k_ref[...] = jnp.dot(xb, wk_ref[...],
                         preferred_element_type=_f32).astype(_bf16)
    v_ref[...] = jnp.dot(xb, wv_ref[...],
                         preferred_element_type=_f32).astype(_bf16)


def _qkv(x, g1, wq, wk, wv):
    out = jax.ShapeDtypeStruct((S, D), _bf16)
    return pl.pallas_call(
        _k1_body,
        grid=(NQ,),
        in_specs=[
            pl.BlockSpec((QB, D), lambda i: (i, 0)),
            pl.BlockSpec((1, D), lambda i: (0, 0)),
            pl.BlockSpec((D, D), lambda i: (0, 0)),
            pl.BlockSpec((D, D), lambda i: (0, 0)),
            pl.BlockSpec((D, D), lambda i: (0, 0)),
        ],
        out_specs=[
            pl.BlockSpec((QB, D), lambda i: (i, 0)),
            pl.BlockSpec((QB, D), lambda i: (i, 0)),
            pl.BlockSpec((QB, D), lambda i: (i, 0)),
        ],
        out_shape=[out, out, out],
    )(x, g1, wq, wk, wv)


def _k2_body(q_ref, k_ref, v_ref, o_ref, s_scr):
    q2 = q_ref[...]
    qb = pl.program_id(1)
    nch = qb + 1
    row = qb * QB + jax.lax.broadcasted_iota(jnp.int32, (QB, QB), 0)
    outs = []
    for hh in range(2):
        q = q2[:, hh * DH:(hh + 1) * DH]

        def p1(kc, m):
            k = k_ref[pl.ds(kc * QB, QB), hh * DH:(hh + 1) * DH]
            s = jax.lax.dot_general(q, k, (((1,), (1,)), ((), ())),
                                    preferred_element_type=_f32)
            s = s * (1.0 / (DH ** 0.5))
            col = kc * QB + jax.lax.broadcasted_iota(jnp.int32, (QB, QB), 1)
            s = jnp.where(row >= col, s, -1e9)
            s_scr[kc] = s
            return jnp.maximum(m, jnp.max(s, axis=1, keepdims=True))

        m = jax.lax.fori_loop(0, nch, p1, jnp.full((QB, 1), -jnp.inf, _f32))

        def p2(kc, z):
            e = jnp.exp(s_scr[kc] - m)
            s_scr[kc] = e
            return z + jnp.sum(e, axis=1, keepdims=True)

        z = jax.lax.fori_loop(0, nch, p2, jnp.zeros((QB, 1), _f32))

        def p3(kc, acc):
            pb = (s_scr[kc] / z).astype(_bf16)
            v = v_ref[pl.ds(kc * QB, QB), hh * DH:(hh + 1) * DH]
            return acc + jnp.dot(pb, v, preferred_element_type=_f32)

        acc = jax.lax.fori_loop(0, nch, p3, jnp.zeros((QB, DH), _f32))
        outs.append(acc)
    o_ref[...] = jnp.concatenate(outs, axis=1).astype(_bf16)


def _attention(q, k, v):
    return pl.pallas_call(
        _k2_body,
        grid=(H // 2, NQ),
        in_specs=[
            pl.BlockSpec((QB, 2 * DH), lambda h, i: (i, h)),
            pl.BlockSpec((S, 2 * DH), lambda h, i: (0, h)),
            pl.BlockSpec((S, 2 * DH), lambda h, i: (0, h)),
        ],
        out_specs=pl.BlockSpec((QB, 2 * DH), lambda h, i: (i, h)),
        out_shape=jax.ShapeDtypeStruct((S, D), _bf16),
        scratch_shapes=[pltpu.VMEM((NQ, QB, QB), _f32)],
    )(q, k, v)


def _k3_body(a_ref, wo_ref, hid_ref, g_ref, gw_ref, h2_ref, xn_ref, lg_ref):
    o = jnp.dot(a_ref[...], wo_ref[...], preferred_element_type=_f32)
    h2 = hid_ref[...] + o
    h2_ref[...] = h2
    var = jnp.mean(h2 * h2, axis=1, keepdims=True)
    xn = (h2 * jax.lax.rsqrt(var + EPS)) * g_ref[...]
    xn_ref[...] = xn
    lg_ref[...] = jnp.dot(xn.astype(_bf16), gw_ref[...],
                          preferred_element_type=_f32)


def _post_attn(attn, wo, hidden, g2, gw_pad):
    return pl.pallas_call(
        _k3_body,
        grid=(NQ,),
        in_specs=[
            pl.BlockSpec((QB, D), lambda i: (i, 0)),
            pl.BlockSpec((D, D), lambda i: (0, 0)),
            pl.BlockSpec((QB, D), lambda i: (i, 0)),
            pl.BlockSpec((1, D), lambda i: (0, 0)),
            pl.BlockSpec((D, 128), lambda i: (0, 0)),
        ],
        out_specs=[
            pl.BlockSpec((QB, D), lambda i: (i, 0)),
            pl.BlockSpec((QB, D), lambda i: (i, 0)),
            pl.BlockSpec((QB, 128), lambda i: (i, 0)),
        ],
        out_shape=[
            jax.ShapeDtypeStruct((S, D), _f32),
            jax.ShapeDtypeStruct((S, D), _f32),
            jax.ShapeDtypeStruct((S, 128), _f32),
        ],
    )(attn, wo, hidden, g2, gw_pad)


def _k4_body(lg_ref, wts_ref, pos_ref, bexp_ref):
    lane = jax.lax.broadcasted_iota(jnp.int32, (S, 128), 1)
    logits = jnp.where(lane < E, lg_ref[...], -1e30)
    m = jnp.max(logits, axis=1, keepdims=True)
    p = jnp.exp(logits - m)
    z = jnp.sum(p, axis=1, keepdims=True)
    probs = p / z
    m1 = jnp.max(probs, axis=1, keepdims=True)
    e1 = jnp.min(jnp.where(probs == m1, lane, 127), axis=1, keepdims=True)
    p2 = jnp.where(lane == e1, -1.0, probs)
    m2 = jnp.max(p2, axis=1, keepdims=True)
    e2 = jnp.min(jnp.where(p2 == m2, lane, 127), axis=1, keepdims=True)
    sw = m1 + m2
    wts_ref[...] = jnp.concatenate([m1 / sw, m2 / sw], axis=1)

    lane8 = jax.lax.broadcasted_iota(jnp.int32, (S, E), 1)
    oh1 = (lane8 == e1).astype(_f32)
    oh2 = (lane8 == e2).astype(_f32)
    onehot = jnp.concatenate([oh1, oh2], axis=0)

    cnt = jnp.sum(onehot, axis=0, keepdims=True)
    cnt_pad = ((cnt.astype(jnp.int32) + (BLK - 1)) // BLK) * BLK
    r8 = jax.lax.broadcasted_iota(jnp.int32, (E, E), 0)
    c8 = jax.lax.broadcasted_iota(jnp.int32, (E, E), 1)
    tri8 = (r8 < c8).astype(_f32)
    offs = jnp.dot(cnt_pad.astype(_f32), tri8,
                   preferred_element_type=_f32)

    inc = onehot
    sh = 1
    while sh < K * S:
        inc = inc + jnp.concatenate(
            [jnp.zeros((sh, E), _f32), inc[:-sh, :]], axis=0)
        sh *= 2
    rank = inc - onehot
    pos = jnp.sum(onehot * (rank + offs), axis=1, keepdims=True)
    posq = 4 * pos.astype(jnp.int32)
    pos_ref[...] = jnp.concatenate(
        [posq, posq + 1, posq + 2, posq + 3], axis=1)

    used = jnp.sum(cnt_pad.astype(_f32)) * (1.0 / BLK)
    brow = jax.lax.broadcasted_iota(jnp.int32, (128, E), 0)
    boffs = jnp.broadcast_to(offs * (1.0 / BLK), (128, E))
    bexp = jnp.sum((brow.astype(_f32) >= boffs).astype(_f32),
                   axis=1, keepdims=True) - 1.0
    brow1 = jax.lax.broadcasted_iota(jnp.int32, (128, 1), 0)
    bexp = jnp.where(brow1.astype(_f32) < used, bexp, -1.0)
    bexp_ref[...] = bexp.astype(jnp.int32)


def _routing(logits):
    return pl.pallas_call(
        _k4_body,
        out_shape=[
            jax.ShapeDtypeStruct((S, K), _f32),
            jax.ShapeDtypeStruct((K * S, 4), jnp.int32),
            jax.ShapeDtypeStruct((128, 1), jnp.int32),
        ],
    )(logits)


def _sc_mesh():
    return plsc.VectorSubcoreMesh(core_axis_name="core",
                                  subcore_axis_name="subcore")


def _dispatch_sc(xn2_q, idx2):
    @pl.kernel(out_type=jax.ShapeDtypeStruct((4 * PBUF, SCC), _f32),
               mesh=_sc_mesh(), scratch_types=[])
    def k(x_hbm, i_hbm, o_hbm):
        def body(x_vmem, i_vmem):
            pltpu.sync_copy(x_vmem, o_hbm.at[i_vmem.at[0]])
        pltpu.emit_pipeline(
            body,
            grid=(K, 4 * S // SCW),
            in_specs=[
                pl.BlockSpec((SCW, SCC), index_map=lambda kk, j: (j, 0)),
                pl.BlockSpec((1, SCW), index_map=lambda kk, j: (kk, j)),
            ],
            out_specs=[],
            core_axis_name=("core", "subcore"),
            dimension_semantics=(pltpu.PARALLEL, pltpu.PARALLEL),
        )(x_hbm, i_hbm)
    return k(xn2_q, idx2)


def _combine_sc(out_sorted_q, pos_flat):
    @pl.kernel(out_type=jax.ShapeDtypeStruct((4 * K * S, SCC), _f32),
               mesh=_sc_mesh(), scratch_types=[])
    def k(d_hbm, i_hbm, o_hbm):
        def body(i_vmem, o_vmem):
            pltpu.sync_copy(d_hbm.at[i_vmem.at[0]], o_vmem)
        pltpu.emit_pipeline(
            body,
            grid=(4 * K * S // SCW,),
            in_specs=[pl.BlockSpec((1, SCW), index_map=lambda j: (0, j))],
            out_specs=[pl.BlockSpec((SCW, SCC), index_map=lambda j: (j, 0))],
            core_axis_name=("core", "subcore"),
            dimension_semantics=(pltpu.PARALLEL,),
        )(i_hbm, o_hbm)
    return k(out_sorted_q, pos_flat)


def _k6_body(bexp_ref, x_ref, w1_ref, w2_ref, o_ref, w1b, w2b):
    i = pl.program_id(0)
    e = bexp_ref[i]
    eprev = jnp.where(i > 0, bexp_ref[jnp.maximum(i - 1, 0)], -2)

    @pl.when((e >= 0) & (e != eprev))
    def _():
        w1b[...] = w1_ref[0].astype(_bf16)
        w2b[...] = w2_ref[0].astype(_bf16)

    @pl.when(e >= 0)
    def _():
        x = x_ref[...].astype(_bf16)
        h1 = jnp.dot(x, w1b[...], preferred_element_type=_f32)
        g = h1[:, :FF]
        u = h1[:, FF:]
        act = (g * jax.lax.logistic(g) * u).astype(_bf16)
        o_ref[...] = jnp.dot(act, w2b[...], preferred_element_type=_f32)


def _expert_ffn(bexp, x_sorted, w1, w2):
    def wmap(i, s):
        return (jnp.maximum(s[i], 0), 0, 0)

    grid_spec = pltpu.PrefetchScalarGridSpec(
        num_scalar_prefetch=1,
        grid=(NB,),
        in_specs=[
            pl.BlockSpec((BLK, D), lambda i, s: (i, 0)),
            pl.BlockSpec((1, D, 2 * FF), wmap),
            pl.BlockSpec((1, FF, D), wmap),
        ],
        out_specs=pl.BlockSpec((BLK, D), lambda i, s: (i, 0)),
        scratch_shapes=[
            pltpu.VMEM((D, 2 * FF), _bf16),
            pltpu.VMEM((FF, D), _bf16),
        ],
    )
    return pl.pallas_call(
        _k6_body,
        grid_spec=grid_spec,
        out_shape=jax.ShapeDtypeStruct((PBUF, D), _f32),
    )(bexp, x_sorted, w1, w2)


def _k8_body(h2_ref, g0_ref, g1_ref, w_ref, o_ref):
    w0 = w_ref[:, 0:1]
    w1c = w_ref[:, 1:2]
    o_ref[...] = h2_ref[...] + w0 * g0_ref[...] + w1c * g1_ref[...]


def _final(h2, g_all, wts):
    return pl.pallas_call(
        _k8_body,
        grid=(NQ,),
        in_specs=[
            pl.BlockSpec((QB, D), lambda i: (i, 0)),
            pl.BlockSpec((QB, D), lambda i: (i, 0)),
            pl.BlockSpec((QB, D), lambda i: (NQ + i, 0)),
            pl.BlockSpec((QB, K), lambda i: (i, 0)),
        ],
        out_specs=pl.BlockSpec((QB, D), lambda i: (i, 0)),
        out_shape=jax.ShapeDtypeStruct((S, D), _f32),
    )(h2, g_all, g_all, wts)


def kernel(hidden_states, ln1_gamma, ln2_gamma, wq, wk, wv, wo, gate_w, w1, w2):
    gw_pad = jnp.pad(gate_w, ((0, 0), (0, 128 - E))).astype(_bf16)
    g1 = ln1_gamma.reshape(1, D)
    g2 = ln2_gamma.reshape(1, D)

    q, k, v = _qkv(hidden_states, g1, wq.astype(_bf16), wk.astype(_bf16),
                   wv.astype(_bf16))
    attn = _attention(q, k, v)
    h2, xn2, logits = _post_attn(attn, wo.astype(_bf16), hidden_states, g2,
                                 gw_pad)
    wts, pos4, bexp = _routing(logits)

    pos_flat = pos4.reshape(1, 4 * K * S)
    idx2 = pos4.reshape(K, 4 * S)
    bexp_flat = bexp.reshape(-1)[:NB]

    x_sorted_q = _dispatch_sc(xn2.reshape(4 * S, SCC), idx2)
    out_sorted = _expert_ffn(bexp_flat, x_sorted_q.reshape(PBUF, D), w1, w2)
    g_q = _combine_sc(out_sorted.reshape(4 * PBUF, SCC), pos_flat)

    return _final(h2, g_q.reshape(K * S, D), wts)


# speedup vs baseline: 1.0433x; 1.0433x over previous
import jax
import jax.numpy as jnp
from jax.experimental import pallas as pl
from jax.experimental.pallas import tpu as pltpu
from jax.experimental.pallas import tpu_sc as plsc

S = 2048
D = 1024
H = 16
DH = D // H
FF = 512
E = 8
K = 2
EPS = 1e-6

QB = 512
NQ = S // QB
BLK = 256
PBUF = S * K + E * BLK
NB = PBUF // BLK
SCW = 128
SCC = D // 4

_f32 = jnp.float32
_bf16 = jnp.bfloat16


def _k1_body(x_ref, g_ref, wq_ref, wk_ref, wv_ref, q_ref, k_ref, v_ref):
    x = x_ref[...]
    var = jnp.mean(x * x, axis=1, keepdims=True)
    xn = (x * jax.lax.rsqrt(var + EPS)) * g_ref[...]
    xb = xn.astype(_bf16)
    q_ref[...] = jnp.dot(xb, wq_ref[...],
                         preferred_element_type=_f32).astype(_bf16)
    k_ref[...] = jnp.dot(xb, wk_ref[...],
                         preferred_element_type=_f32).astype(_bf16)
    v_ref[...] = jnp.dot(xb, wv_ref[...],
                         preferred_element_type=_f32).astype(_bf16)


def _qkv(x, g1, wq, wk, wv):
    out = jax.ShapeDtypeStruct((S, D), _bf16)
    return pl.pallas_call(
        _k1_body,
        grid=(NQ,),
        in_specs=[
            pl.BlockSpec((QB, D), lambda i: (i, 0)),
            pl.BlockSpec((1, D), lambda i: (0, 0)),
            pl.BlockSpec((D, D), lambda i: (0, 0)),
            pl.BlockSpec((D, D), lambda i: (0, 0)),
            pl.BlockSpec((D, D), lambda i: (0, 0)),
        ],
        out_specs=[
            pl.BlockSpec((QB, D), lambda i: (i, 0)),
            pl.BlockSpec((QB, D), lambda i: (i, 0)),
            pl.BlockSpec((QB, D), lambda i: (i, 0)),
        ],
        out_shape=[out, out, out],
    )(x, g1, wq, wk, wv)


def _k2_body(q_ref, k_ref, v_ref, o_ref):
    q2 = q_ref[...]
    k2 = k_ref[...]
    v2 = v_ref[...]
    qb = pl.program_id(1)
    row = qb * QB + jax.lax.broadcasted_iota(jnp.int32, (QB, S), 0)
    col = jax.lax.broadcasted_iota(jnp.int32, (QB, S), 1)
    causal = row >= col
    outs = []
    for hh in range(2):
        q = q2[:, hh * DH:(hh + 1) * DH]
        k = k2[:, hh * DH:(hh + 1) * DH]
        v = v2[:, hh * DH:(hh + 1) * DH]
        s = jax.lax.dot_general(q, k, (((1,), (1,)), ((), ())),
                                preferred_element_type=_f32)
        s = s * (1.0 / (DH ** 0.5))
        s = jnp.where(causal, s, -1e9)
        m = jnp.max(s, axis=1, keepdims=True)
        p = jnp.exp(s - m)
        z = jnp.sum(p, axis=1, keepdims=True)
        pb = (p / z).astype(_bf16)
        outs.append(jnp.dot(pb, v, preferred_element_type=_f32))
    o_ref[...] = jnp.concatenate(outs, axis=1).astype(_bf16)


def _attention(q, k, v):
    return pl.pallas_call(
        _k2_body,
        grid=(H // 2, NQ),
        in_specs=[
            pl.BlockSpec((QB, 2 * DH), lambda h, i: (i, h)),
            pl.BlockSpec((S, 2 * DH), lambda h, i: (0, h)),
            pl.BlockSpec((S, 2 * DH), lambda h, i: (0, h)),
        ],
        out_specs=pl.BlockSpec((QB, 2 * DH), lambda h, i: (i, h)),
        out_shape=jax.ShapeDtypeStruct((S, D), _bf16),
    )(q, k, v)


def _k3_body(a_ref, wo_ref, hid_ref, g_ref, gw_ref, h2_ref, xn_ref, lg_ref):
    o = jnp.dot(a_ref[...], wo_ref[...], preferred_element_type=_f32)
    h2 = hid_ref[...] + o
    h2_ref[...] = h2
    var = jnp.mean(h2 * h2, axis=1, keepdims=True)
    xn = (h2 * jax.lax.rsqrt(var + EPS)) * g_ref[...]
    xn_ref[...] = xn
    lg_ref[...] = jnp.dot(xn.astype(_bf16), gw_ref[...],
                          preferred_element_type=_f32)


def _post_attn(attn, wo, hidden, g2, gw_pad):
    return pl.pallas_call(
        _k3_body,
        grid=(NQ,),
        in_specs=[
            pl.BlockSpec((QB, D), lambda i: (i, 0)),
            pl.BlockSpec((D, D), lambda i: (0, 0)),
            pl.BlockSpec((QB, D), lambda i: (i, 0)),
            pl.BlockSpec((1, D), lambda i: (0, 0)),
            pl.BlockSpec((D, 128), lambda i: (0, 0)),
        ],
        out_specs=[
            pl.BlockSpec((QB, D), lambda i: (i, 0)),
            pl.BlockSpec((QB, D), lambda i: (i, 0)),
            pl.BlockSpec((QB, 128), lambda i: (i, 0)),
        ],
        out_shape=[
            jax.ShapeDtypeStruct((S, D), _f32),
            jax.ShapeDtypeStruct((S, D), _f32),
            jax.ShapeDtypeStruct((S, 128), _f32),
        ],
    )(attn, wo, hidden, g2, gw_pad)


def _k4_body(lg_ref, wts_ref, pos_ref, bexp_ref):
    lane = jax.lax.broadcasted_iota(jnp.int32, (S, 128), 1)
    logits = jnp.where(lane < E, lg_ref[...], -1e30)
    m = jnp.max(logits, axis=1, keepdims=True)
    p = jnp.exp(logits - m)
    z = jnp.sum(p, axis=1, keepdims=True)
    probs = p / z
    m1 = jnp.max(probs, axis=1, keepdims=True)
    e1 = jnp.min(jnp.where(probs == m1, lane, 127), axis=1, keepdims=True)
    p2 = jnp.where(lane == e1, -1.0, probs)
    m2 = jnp.max(p2, axis=1, keepdims=True)
    e2 = jnp.min(jnp.where(p2 == m2, lane, 127), axis=1, keepdims=True)
    sw = m1 + m2
    wts_ref[...] = jnp.concatenate([m1 / sw, m2 / sw], axis=1)

    lane8 = jax.lax.broadcasted_iota(jnp.int32, (S, E), 1)
    oh1 = (lane8 == e1).astype(_f32)
    oh2 = (lane8 == e2).astype(_f32)
    onehot = jnp.concatenate([oh1, oh2], axis=0)

    cnt = jnp.sum(onehot, axis=0, keepdims=True)
    cnt_pad = ((cnt.astype(jnp.int32) + (BLK - 1)) // BLK) * BLK
    r8 = jax.lax.broadcasted_iota(jnp.int32, (E, E), 0)
    c8 = jax.lax.broadcasted_iota(jnp.int32, (E, E), 1)
    tri8 = (r8 < c8).astype(_f32)
    offs = jnp.dot(cnt_pad.astype(_f32), tri8,
                   preferred_element_type=_f32)

    inc = onehot
    sh = 1
    while sh < K * S:
        inc = inc + jnp.concatenate(
            [jnp.zeros((sh, E), _f32), inc[:-sh, :]], axis=0)
        sh *= 2
    rank = inc - onehot
    pos = jnp.sum(onehot * (rank + offs), axis=1, keepdims=True)
    posq = 4 * pos.astype(jnp.int32)
    pos_ref[...] = jnp.concatenate(
        [posq, posq + 1, posq + 2, posq + 3], axis=1)

    used = jnp.sum(cnt_pad.astype(_f32)) * (1.0 / BLK)
    brow = jax.lax.broadcasted_iota(jnp.int32, (128, E), 0)
    boffs = jnp.broadcast_to(offs * (1.0 / BLK), (128, E))
    bexp = jnp.sum((brow.astype(_f32) >= boffs).astype(_f32),
                   axis=1, keepdims=True) - 1.0
    brow1 = jax.lax.broadcasted_iota(jnp.int32, (128, 1), 0)
    bexp = jnp.where(brow1.astype(_f32) < used, bexp, -1.0)
    bexp_ref[...] = bexp.astype(jnp.int32)


def _routing(logits):
    return pl.pallas_call(
        _k4_body,
        out_shape=[
            jax.ShapeDtypeStruct((S, K), _f32),
            jax.ShapeDtypeStruct((K * S, 4), jnp.int32),
            jax.ShapeDtypeStruct((128, 1), jnp.int32),
        ],
    )(logits)


def _sc_mesh():
    return plsc.VectorSubcoreMesh(core_axis_name="core",
                                  subcore_axis_name="subcore")


def _dispatch_sc(xn2_q, idx2):
    @pl.kernel(out_type=jax.ShapeDtypeStruct((4 * PBUF, SCC), _f32),
               mesh=_sc_mesh(), scratch_types=[])
    def k(x_hbm, i_hbm, o_hbm):
        def body(x_vmem, i_vmem):
            pltpu.sync_copy(x_vmem, o_hbm.at[i_vmem.at[0]])
        pltpu.emit_pipeline(
            body,
            grid=(K, 4 * S // SCW),
            in_specs=[
                pl.BlockSpec((SCW, SCC), index_map=lambda kk, j: (j, 0)),
                pl.BlockSpec((1, SCW), index_map=lambda kk, j: (kk, j)),
            ],
            out_specs=[],
            core_axis_name=("core", "subcore"),
            dimension_semantics=(pltpu.PARALLEL, pltpu.PARALLEL),
        )(x_hbm, i_hbm)
    return k(xn2_q, idx2)


def _combine_sc(out_sorted_q, pos_flat):
    @pl.kernel(out_type=jax.ShapeDtypeStruct((4 * K * S, SCC), _f32),
               mesh=_sc_mesh(), scratch_types=[])
    def k(d_hbm, i_hbm, o_hbm):
        def body(i_vmem, o_vmem):
            pltpu.sync_copy(d_hbm.at[i_vmem.at[0]], o_vmem)
        pltpu.emit_pipeline(
            body,
            grid=(4 * K * S // SCW,),
            in_specs=[pl.BlockSpec((1, SCW), index_map=lambda j: (0, j))],
            out_specs=[pl.BlockSpec((SCW, SCC), index_map=lambda j: (j, 0))],
            core_axis_name=("core", "subcore"),
            dimension_semantics=(pltpu.PARALLEL,),
        )(i_hbm, o_hbm)
    return k(out_sorted_q, pos_flat)


def _k6_body(bexp_ref, x_ref, w1_ref, w2_ref, o_ref, w1b, w2b):
    i = pl.program_id(0)
    e = bexp_ref[i]
    eprev = jnp.where(i > 0, bexp_ref[jnp.maximum(i - 1, 0)], -2)

    @pl.when((e >= 0) & (e != eprev))
    def _():
        w1b[...] = w1_ref[0].astype(_bf16)
        w2b[...] = w2_ref[0].astype(_bf16)

    @pl.when(e >= 0)
    def _():
        x = x_ref[...].astype(_bf16)
        h1 = jnp.dot(x, w1b[...], preferred_element_type=_f32)
        g = h1[:, :FF]
        u = h1[:, FF:]
        act = (g * jax.lax.logistic(g) * u).astype(_bf16)
        o_ref[...] = jnp.dot(act, w2b[...], preferred_element_type=_f32)


def _expert_ffn(bexp, x_sorted, w1, w2):
    def wmap(i, s):
        return (jnp.maximum(s[i], 0), 0, 0)

    grid_spec = pltpu.PrefetchScalarGridSpec(
        num_scalar_prefetch=1,
        grid=(NB,),
        in_specs=[
            pl.BlockSpec((BLK, D), lambda i, s: (i, 0)),
            pl.BlockSpec((1, D, 2 * FF), wmap),
            pl.BlockSpec((1, FF, D), wmap),
        ],
        out_specs=pl.BlockSpec((BLK, D), lambda i, s: (i, 0)),
        scratch_shapes=[
            pltpu.VMEM((D, 2 * FF), _bf16),
            pltpu.VMEM((FF, D), _bf16),
        ],
    )
    return pl.pallas_call(
        _k6_body,
        grid_spec=grid_spec,
        out_shape=jax.ShapeDtypeStruct((PBUF, D), _f32),
    )(bexp, x_sorted, w1, w2)


def _k8_body(h2_ref, g0_ref, g1_ref, w_ref, o_ref):
    w0 = w_ref[:, 0:1]
    w1c = w_ref[:, 1:2]
    o_ref[...] = h2_ref[...] + w0 * g0_ref[...] + w1c * g1_ref[...]


def _final(h2, g_all, wts):
    return pl.pallas_call(
        _k8_body,
        grid=(NQ,),
        in_specs=[
            pl.BlockSpec((QB, D), lambda i: (i, 0)),
            pl.BlockSpec((QB, D), lambda i: (i, 0)),
            pl.BlockSpec((QB, D), lambda i: (NQ + i, 0)),
            pl.BlockSpec((QB, K), lambda i: (i, 0)),
        ],
        out_specs=pl.BlockSpec((QB, D), lambda i: (i, 0)),
        out_shape=jax.ShapeDtypeStruct((S, D), _f32),
    )(h2, g_all, g_all, wts)


def kernel(hidden_states, ln1_gamma, ln2_gamma, wq, wk, wv, wo, gate_w, w1, w2):
    gw_pad = jnp.pad(gate_w, ((0, 0), (0, 128 - E))).astype(_bf16)
    g1 = ln1_gamma.reshape(1, D)
    g2 = ln2_gamma.reshape(1, D)

    q, k, v = _qkv(hidden_states, g1, wq.astype(_bf16), wk.astype(_bf16),
                   wv.astype(_bf16))
    attn = _attention(q, k, v)
    h2, xn2, logits = _post_attn(attn, wo.astype(_bf16), hidden_states, g2,
                                 gw_pad)
    wts, pos4, bexp = _routing(logits)

    pos_flat = pos4.reshape(1, 4 * K * S)
    idx2 = pos4.reshape(K, 4 * S)
    bexp_flat = bexp.reshape(-1)[:NB]

    x_sorted_q = _dispatch_sc(xn2.reshape(4 * S, SCC), idx2)
    out_sorted = _expert_ffn(bexp_flat, x_sorted_q.reshape(PBUF, D), w1, w2)
    g_q = _combine_sc(out_sorted.reshape(4 * PBUF, SCC), pos_flat)

    return _final(h2, g_q.reshape(K * S, D), wts)


# speedup vs baseline: 1.2098x; 1.1596x over previous
import jax
import jax.numpy as jnp
from jax.experimental import pallas as pl
from jax.experimental.pallas import tpu as pltpu
from jax.experimental.pallas import tpu_sc as plsc

S = 2048
D = 1024
H = 16
DH = D // H
FF = 512
E = 8
K = 2
EPS = 1e-6

QB = 512
NQ = S // QB
BLK = 256
PBUF = S * K + E * BLK
NB = PBUF // BLK
SCW = 128
SCC = D // 4

_f32 = jnp.float32
_bf16 = jnp.bfloat16


def _k1_body(x_ref, g_ref, wq_ref, wk_ref, wv_ref, q_ref, k_ref, v_ref):
    x = x_ref[...]
    var = jnp.mean(x * x, axis=1, keepdims=True)
    xn = (x * jax.lax.rsqrt(var + EPS)) * g_ref[...]
    xb = xn.astype(_bf16)
    q_ref[...] = jnp.dot(xb, wq_ref[...],
                         preferred_element_type=_f32).astype(_bf16)
    k_ref[...] = jnp.dot(xb, wk_ref[...],
                         preferred_element_type=_f32).astype(_bf16)
    v_ref[...] = jnp.dot(xb, wv_ref[...],
                         preferred_element_type=_f32).astype(_bf16)


def _qkv(x, g1, wq, wk, wv):
    out = jax.ShapeDtypeStruct((S, D), _bf16)
    return pl.pallas_call(
        _k1_body,
        grid=(NQ,),
        in_specs=[
            pl.BlockSpec((QB, D), lambda i: (i, 0)),
            pl.BlockSpec((1, D), lambda i: (0, 0)),
            pl.BlockSpec((D, D), lambda i: (0, 0)),
            pl.BlockSpec((D, D), lambda i: (0, 0)),
            pl.BlockSpec((D, D), lambda i: (0, 0)),
        ],
        out_specs=[
            pl.BlockSpec((QB, D), lambda i: (i, 0)),
            pl.BlockSpec((QB, D), lambda i: (i, 0)),
            pl.BlockSpec((QB, D), lambda i: (i, 0)),
        ],
        out_shape=[out, out, out],
    )(x, g1, wq, wk, wv)


def _k2_body(qb, skv, q_ref, k_ref, v_ref, a_ref, o_ref):
    del a_ref
    q2 = q_ref[...]
    k2 = k_ref[...]
    v2 = v_ref[...]
    row = qb * QB + jax.lax.broadcasted_iota(jnp.int32, (QB, skv), 0)
    col = jax.lax.broadcasted_iota(jnp.int32, (QB, skv), 1)
    causal = row >= col
    outs = []
    for hh in range(2):
        q = q2[:, hh * DH:(hh + 1) * DH]
        k = k2[:, hh * DH:(hh + 1) * DH]
        v = v2[:, hh * DH:(hh + 1) * DH]
        s = jax.lax.dot_general(q, k, (((1,), (1,)), ((), ())),
                                preferred_element_type=_f32)
        s = s * (1.0 / (DH ** 0.5))
        s = jnp.where(causal, s, -1e9)
        m = jnp.max(s, axis=1, keepdims=True)
        p = jnp.exp(s - m)
        z = jnp.sum(p, axis=1, keepdims=True)
        pb = (p / z).astype(_bf16)
        outs.append(jnp.dot(pb, v, preferred_element_type=_f32))
    o_ref[...] = jnp.concatenate(outs, axis=1).astype(_bf16)


def _attention(q, k, v):
    import functools
    a = jnp.zeros((S, D), _bf16)
    for qb in range(NQ):
        skv = (qb + 1) * QB
        a = pl.pallas_call(
            functools.partial(_k2_body, qb, skv),
            grid=(H // 2,),
            in_specs=[
                pl.BlockSpec((QB, 2 * DH), lambda h, _qb=qb: (_qb, h)),
                pl.BlockSpec((skv, 2 * DH), lambda h: (0, h)),
                pl.BlockSpec((skv, 2 * DH), lambda h: (0, h)),
                pl.BlockSpec((QB, 2 * DH), lambda h, _qb=qb: (_qb, h)),
            ],
            out_specs=pl.BlockSpec((QB, 2 * DH), lambda h, _qb=qb: (_qb, h)),
            out_shape=jax.ShapeDtypeStruct((S, D), _bf16),
            input_output_aliases={3: 0},
        )(q, k, v, a)
    return a


def _k3_body(a_ref, wo_ref, hid_ref, g_ref, gw_ref, h2_ref, xn_ref, lg_ref):
    o = jnp.dot(a_ref[...], wo_ref[...], preferred_element_type=_f32)
    h2 = hid_ref[...] + o
    h2_ref[...] = h2
    var = jnp.mean(h2 * h2, axis=1, keepdims=True)
    xn = (h2 * jax.lax.rsqrt(var + EPS)) * g_ref[...]
    xn_ref[...] = xn
    lg_ref[...] = jnp.dot(xn.astype(_bf16), gw_ref[...],
                          preferred_element_type=_f32)


def _post_attn(attn, wo, hidden, g2, gw_pad):
    return pl.pallas_call(
        _k3_body,
        grid=(NQ,),
        in_specs=[
            pl.BlockSpec((QB, D), lambda i: (i, 0)),
            pl.BlockSpec((D, D), lambda i: (0, 0)),
            pl.BlockSpec((QB, D), lambda i: (i, 0)),
            pl.BlockSpec((1, D), lambda i: (0, 0)),
            pl.BlockSpec((D, 128), lambda i: (0, 0)),
        ],
        out_specs=[
            pl.BlockSpec((QB, D), lambda i: (i, 0)),
            pl.BlockSpec((QB, D), lambda i: (i, 0)),
            pl.BlockSpec((QB, 128), lambda i: (i, 0)),
        ],
        out_shape=[
            jax.ShapeDtypeStruct((S, D), _f32),
            jax.ShapeDtypeStruct((S, D), _f32),
            jax.ShapeDtypeStruct((S, 128), _f32),
        ],
    )(attn, wo, hidden, g2, gw_pad)


def _k4_body(lg_ref, wts_ref, pos_ref, bexp_ref):
    lane = jax.lax.broadcasted_iota(jnp.int32, (S, 128), 1)
    logits = jnp.where(lane < E, lg_ref[...], -1e30)
    m = jnp.max(logits, axis=1, keepdims=True)
    p = jnp.exp(logits - m)
    z = jnp.sum(p, axis=1, keepdims=True)
    probs = p / z
    m1 = jnp.max(probs, axis=1, keepdims=True)
    e1 = jnp.min(jnp.where(probs == m1, lane, 127), axis=1, keepdims=True)
    p2 = jnp.where(lane == e1, -1.0, probs)
    m2 = jnp.max(p2, axis=1, keepdims=True)
    e2 = jnp.min(jnp.where(p2 == m2, lane, 127), axis=1, keepdims=True)
    sw = m1 + m2
    wts_ref[...] = jnp.concatenate([m1 / sw, m2 / sw], axis=1)

    lane8 = jax.lax.broadcasted_iota(jnp.int32, (S, E), 1)
    oh1 = (lane8 == e1).astype(_f32)
    oh2 = (lane8 == e2).astype(_f32)
    onehot = jnp.concatenate([oh1, oh2], axis=0)

    cnt = jnp.sum(onehot, axis=0, keepdims=True)
    cnt_pad = ((cnt.astype(jnp.int32) + (BLK - 1)) // BLK) * BLK
    r8 = jax.lax.broadcasted_iota(jnp.int32, (E, E), 0)
    c8 = jax.lax.broadcasted_iota(jnp.int32, (E, E), 1)
    tri8 = (r8 < c8).astype(_f32)
    offs = jnp.dot(cnt_pad.astype(_f32), tri8,
                   preferred_element_type=_f32)

    inc = onehot
    sh = 1
    while sh < K * S:
        inc = inc + jnp.concatenate(
            [jnp.zeros((sh, E), _f32), inc[:-sh, :]], axis=0)
        sh *= 2
    rank = inc - onehot
    pos = jnp.sum(onehot * (rank + offs), axis=1, keepdims=True)
    posq = 4 * pos.astype(jnp.int32)
    pos_ref[...] = jnp.concatenate(
        [posq, posq + 1, posq + 2, posq + 3], axis=1)

    used = jnp.sum(cnt_pad.astype(_f32)) * (1.0 / BLK)
    brow = jax.lax.broadcasted_iota(jnp.int32, (128, E), 0)
    boffs = jnp.broadcast_to(offs * (1.0 / BLK), (128, E))
    bexp = jnp.sum((brow.astype(_f32) >= boffs).astype(_f32),
                   axis=1, keepdims=True) - 1.0
    brow1 = jax.lax.broadcasted_iota(jnp.int32, (128, 1), 0)
    bexp = jnp.where(brow1.astype(_f32) < used, bexp, -1.0)
    bexp_ref[...] = bexp.astype(jnp.int32)


def _routing(logits):
    return pl.pallas_call(
        _k4_body,
        out_shape=[
            jax.ShapeDtypeStruct((S, K), _f32),
            jax.ShapeDtypeStruct((K * S, 4), jnp.int32),
            jax.ShapeDtypeStruct((128, 1), jnp.int32),
        ],
    )(logits)


def _sc_mesh():
    return plsc.VectorSubcoreMesh(core_axis_name="core",
                                  subcore_axis_name="subcore")


def _dispatch_sc(xn2_q, idx2):
    @pl.kernel(out_type=jax.ShapeDtypeStruct((4 * PBUF, SCC), _f32),
               mesh=_sc_mesh(), scratch_types=[])
    def k(x_hbm, i_hbm, o_hbm):
        def body(x_vmem, i_vmem):
            pltpu.sync_copy(x_vmem, o_hbm.at[i_vmem.at[0]])
        pltpu.emit_pipeline(
            body,
            grid=(K, 4 * S // SCW),
            in_specs=[
                pl.BlockSpec((SCW, SCC), index_map=lambda kk, j: (j, 0)),
                pl.BlockSpec((1, SCW), index_map=lambda kk, j: (kk, j)),
            ],
            out_specs=[],
            core_axis_name=("core", "subcore"),
            dimension_semantics=(pltpu.PARALLEL, pltpu.PARALLEL),
        )(x_hbm, i_hbm)
    return k(xn2_q, idx2)


def _combine_sc(out_sorted_q, pos_flat):
    @pl.kernel(out_type=jax.ShapeDtypeStruct((4 * K * S, SCC), _f32),
               mesh=_sc_mesh(), scratch_types=[])
    def k(d_hbm, i_hbm, o_hbm):
        def body(i_vmem, o_vmem):
            pltpu.sync_copy(d_hbm.at[i_vmem.at[0]], o_vmem)
        pltpu.emit_pipeline(
            body,
            grid=(4 * K * S // SCW,),
            in_specs=[pl.BlockSpec((1, SCW), index_map=lambda j: (0, j))],
            out_specs=[pl.BlockSpec((SCW, SCC), index_map=lambda j: (j, 0))],
            core_axis_name=("core", "subcore"),
            dimension_semantics=(pltpu.PARALLEL,),
        )(i_hbm, o_hbm)
    return k(out_sorted_q, pos_flat)


def _k6_body(bexp_ref, x_ref, w1_ref, w2_ref, o_ref, w1b, w2b):
    i = pl.program_id(0)
    e = bexp_ref[i]
    eprev = jnp.where(i > 0, bexp_ref[jnp.maximum(i - 1, 0)], -2)

    @pl.when((e >= 0) & (e != eprev))
    def _():
        w1b[...] = w1_ref[0].astype(_bf16)
        w2b[...] = w2_ref[0].astype(_bf16)

    @pl.when(e >= 0)
    def _():
        x = x_ref[...].astype(_bf16)
        h1 = jnp.dot(x, w1b[...], preferred_element_type=_f32)
        g = h1[:, :FF]
        u = h1[:, FF:]
        act = (g * jax.lax.logistic(g) * u).astype(_bf16)
        o_ref[...] = jnp.dot(act, w2b[...], preferred_element_type=_f32)


def _expert_ffn(bexp, x_sorted, w1, w2):
    def wmap(i, s):
        return (jnp.maximum(s[i], 0), 0, 0)

    grid_spec = pltpu.PrefetchScalarGridSpec(
        num_scalar_prefetch=1,
        grid=(NB,),
        in_specs=[
            pl.BlockSpec((BLK, D), lambda i, s: (i, 0)),
            pl.BlockSpec((1, D, 2 * FF), wmap),
            pl.BlockSpec((1, FF, D), wmap),
        ],
        out_specs=pl.BlockSpec((BLK, D), lambda i, s: (i, 0)),
        scratch_shapes=[
            pltpu.VMEM((D, 2 * FF), _bf16),
            pltpu.VMEM((FF, D), _bf16),
        ],
    )
    return pl.pallas_call(
        _k6_body,
        grid_spec=grid_spec,
        out_shape=jax.ShapeDtypeStruct((PBUF, D), _f32),
    )(bexp, x_sorted, w1, w2)


def _k8_body(h2_ref, g0_ref, g1_ref, w_ref, o_ref):
    w0 = w_ref[:, 0:1]
    w1c = w_ref[:, 1:2]
    o_ref[...] = h2_ref[...] + w0 * g0_ref[...] + w1c * g1_ref[...]


def _final(h2, g_all, wts):
    return pl.pallas_call(
        _k8_body,
        grid=(NQ,),
        in_specs=[
            pl.BlockSpec((QB, D), lambda i: (i, 0)),
            pl.BlockSpec((QB, D), lambda i: (i, 0)),
            pl.BlockSpec((QB, D), lambda i: (NQ + i, 0)),
            pl.BlockSpec((QB, K), lambda i: (i, 0)),
        ],
        out_specs=pl.BlockSpec((QB, D), lambda i: (i, 0)),
        out_shape=jax.ShapeDtypeStruct((S, D), _f32),
    )(h2, g_all, g_all, wts)


def kernel(hidden_states, ln1_gamma, ln2_gamma, wq, wk, wv, wo, gate_w, w1, w2):
    gw_pad = jnp.pad(gate_w, ((0, 0), (0, 128 - E))).astype(_bf16)
    g1 = ln1_gamma.reshape(1, D)
    g2 = ln2_gamma.reshape(1, D)

    q, k, v = _qkv(hidden_states, g1, wq.astype(_bf16), wk.astype(_bf16),
                   wv.astype(_bf16))
    attn = _attention(q, k, v)
    h2, xn2, logits = _post_attn(attn, wo.astype(_bf16), hidden_states, g2,
                                 gw_pad)
    wts, pos4, bexp = _routing(logits)

    pos_flat = pos4.reshape(1, 4 * K * S)
    idx2 = pos4.reshape(K, 4 * S)
    bexp_flat = bexp.reshape(-1)[:NB]

    x_sorted_q = _dispatch_sc(xn2.reshape(4 * S, SCC), idx2)
    out_sorted = _expert_ffn(bexp_flat, x_sorted_q.reshape(PBUF, D), w1, w2)
    g_q = _combine_sc(out_sorted.reshape(4 * PBUF, SCC), pos_flat)

    return _final(h2, g_q.reshape(K * S, D), wts)


# speedup vs baseline: 1.5543x; 1.2848x over previous
import jax
import jax.numpy as jnp
from jax.experimental import pallas as pl
from jax.experimental.pallas import tpu as pltpu
from jax.experimental.pallas import tpu_sc as plsc

S = 2048
D = 1024
H = 16
DH = D // H
FF = 512
E = 8
K = 2
EPS = 1e-6

QB = 512
NQ = S // QB
BLK = 256
PBUF = S * K + E * BLK
NB = PBUF // BLK
SCW = 128
SCC = D // 4

_f32 = jnp.float32
_bf16 = jnp.bfloat16


def _k1_body(x_ref, g_ref, wq_ref, wk_ref, wv_ref, q_ref, k_ref, v_ref):
    x = x_ref[...]
    var = jnp.mean(x * x, axis=1, keepdims=True)
    xn = (x * jax.lax.rsqrt(var + EPS)) * g_ref[...]
    xb = xn.astype(_bf16)
    q_ref[...] = jnp.dot(xb, wq_ref[...],
                         preferred_element_type=_f32).astype(_bf16)
    k_ref[...] = jnp.dot(xb, wk_ref[...],
                         preferred_element_type=_f32).astype(_bf16)
    v_ref[...] = jnp.dot(xb, wv_ref[...],
                         preferred_element_type=_f32).astype(_bf16)


def _qkv(x, g1, wq, wk, wv):
    out = jax.ShapeDtypeStruct((S, D), _bf16)
    return pl.pallas_call(
        _k1_body,
        grid=(NQ,),
        in_specs=[
            pl.BlockSpec((QB, D), lambda i: (i, 0)),
            pl.BlockSpec((1, D), lambda i: (0, 0)),
            pl.BlockSpec((D, D), lambda i: (0, 0)),
            pl.BlockSpec((D, D), lambda i: (0, 0)),
            pl.BlockSpec((D, D), lambda i: (0, 0)),
        ],
        out_specs=[
            pl.BlockSpec((QB, D), lambda i: (i, 0)),
            pl.BlockSpec((QB, D), lambda i: (i, 0)),
            pl.BlockSpec((QB, D), lambda i: (i, 0)),
        ],
        out_shape=[out, out, out],
    )(x, g1, wq, wk, wv)


def _k2_body(qb, skv, q_ref, k_ref, v_ref, a_ref, o_ref):
    del a_ref
    q2 = q_ref[...]
    k2 = k_ref[...]
    v2 = v_ref[...]
    row = qb * QB + jax.lax.broadcasted_iota(jnp.int32, (QB, skv), 0)
    col = jax.lax.broadcasted_iota(jnp.int32, (QB, skv), 1)
    causal = row >= col
    outs = []
    for hh in range(2):
        q = q2[:, hh * DH:(hh + 1) * DH]
        k = k2[:, hh * DH:(hh + 1) * DH]
        v = v2[:, hh * DH:(hh + 1) * DH]
        s = jax.lax.dot_general(q, k, (((1,), (1,)), ((), ())),
                                preferred_element_type=_f32)
        s = s * (1.0 / (DH ** 0.5))
        s = jnp.where(causal, s, -1e9)
        m = jnp.max(s, axis=1, keepdims=True)
        p = jnp.exp(s - m)
        z = jnp.sum(p, axis=1, keepdims=True)
        pb = (p / z).astype(_bf16)
        outs.append(jnp.dot(pb, v, preferred_element_type=_f32))
    o_ref[...] = jnp.concatenate(outs, axis=1).astype(_bf16)


def _attention(q, k, v):
    import functools
    a = jnp.zeros((S, D), _bf16)
    for qb in range(NQ):
        skv = (qb + 1) * QB
        a = pl.pallas_call(
            functools.partial(_k2_body, qb, skv),
            grid=(H // 2,),
            in_specs=[
                pl.BlockSpec((QB, 2 * DH), lambda h, _qb=qb: (_qb, h)),
                pl.BlockSpec((skv, 2 * DH), lambda h: (0, h)),
                pl.BlockSpec((skv, 2 * DH), lambda h: (0, h)),
                pl.BlockSpec((QB, 2 * DH), lambda h, _qb=qb: (_qb, h)),
            ],
            out_specs=pl.BlockSpec((QB, 2 * DH), lambda h, _qb=qb: (_qb, h)),
            out_shape=jax.ShapeDtypeStruct((S, D), _bf16),
            input_output_aliases={3: 0},
        )(q, k, v, a)
    return a


def _k3_body(a_ref, wo_ref, hid_ref, g_ref, gw_ref, h2_ref, xn_ref, lg_ref):
    o = jnp.dot(a_ref[...], wo_ref[...], preferred_element_type=_f32)
    h2 = hid_ref[...] + o
    h2_ref[...] = h2
    var = jnp.mean(h2 * h2, axis=1, keepdims=True)
    xn = (h2 * jax.lax.rsqrt(var + EPS)) * g_ref[...]
    xn_ref[...] = xn.reshape(4 * QB, SCC)
    lg_ref[...] = jnp.dot(xn.astype(_bf16), gw_ref[...],
                          preferred_element_type=_f32)


def _post_attn(attn, wo, hidden, g2, gw_pad):
    return pl.pallas_call(
        _k3_body,
        grid=(NQ,),
        in_specs=[
            pl.BlockSpec((QB, D), lambda i: (i, 0)),
            pl.BlockSpec((D, D), lambda i: (0, 0)),
            pl.BlockSpec((QB, D), lambda i: (i, 0)),
            pl.BlockSpec((1, D), lambda i: (0, 0)),
            pl.BlockSpec((D, 128), lambda i: (0, 0)),
        ],
        out_specs=[
            pl.BlockSpec((QB, D), lambda i: (i, 0)),
            pl.BlockSpec((4 * QB, SCC), lambda i: (i, 0)),
            pl.BlockSpec((QB, 128), lambda i: (i, 0)),
        ],
        out_shape=[
            jax.ShapeDtypeStruct((S, D), _f32),
            jax.ShapeDtypeStruct((4 * S, SCC), _f32),
            jax.ShapeDtypeStruct((S, 128), _f32),
        ],
    )(attn, wo, hidden, g2, gw_pad)


def _k4_body(lg_ref, wts_ref, pos_ref, bexp_ref):
    lane = jax.lax.broadcasted_iota(jnp.int32, (S, 128), 1)
    logits = jnp.where(lane < E, lg_ref[...], -1e30)
    m = jnp.max(logits, axis=1, keepdims=True)
    p = jnp.exp(logits - m)
    z = jnp.sum(p, axis=1, keepdims=True)
    probs = p / z
    m1 = jnp.max(probs, axis=1, keepdims=True)
    e1 = jnp.min(jnp.where(probs == m1, lane, 127), axis=1, keepdims=True)
    p2 = jnp.where(lane == e1, -1.0, probs)
    m2 = jnp.max(p2, axis=1, keepdims=True)
    e2 = jnp.min(jnp.where(p2 == m2, lane, 127), axis=1, keepdims=True)
    sw = m1 + m2
    wts_ref[...] = jnp.concatenate([m1 / sw, m2 / sw], axis=1)

    lane8 = jax.lax.broadcasted_iota(jnp.int32, (S, E), 1)
    oh1 = (lane8 == e1).astype(_f32)
    oh2 = (lane8 == e2).astype(_f32)
    onehot = jnp.concatenate([oh1, oh2], axis=0)

    cnt = jnp.sum(onehot, axis=0, keepdims=True)
    cnt_pad = ((cnt.astype(jnp.int32) + (BLK - 1)) // BLK) * BLK
    r8 = jax.lax.broadcasted_iota(jnp.int32, (E, E), 0)
    c8 = jax.lax.broadcasted_iota(jnp.int32, (E, E), 1)
    tri8 = (r8 < c8).astype(_f32)
    offs = jnp.dot(cnt_pad.astype(_f32), tri8,
                   preferred_element_type=_f32)

    inc = onehot
    sh = 1
    while sh < K * S:
        inc = inc + jnp.concatenate(
            [jnp.zeros((sh, E), _f32), inc[:-sh, :]], axis=0)
        sh *= 2
    rank = inc - onehot
    pos = jnp.sum(onehot * (rank + offs), axis=1, keepdims=True)
    posq = 4 * pos.astype(jnp.int32)
    pos_ref[...] = jnp.concatenate(
        [posq, posq + 1, posq + 2, posq + 3], axis=1)

    used = jnp.sum(cnt_pad.astype(_f32)) * (1.0 / BLK)
    brow = jax.lax.broadcasted_iota(jnp.int32, (128, E), 0)
    boffs = jnp.broadcast_to(offs * (1.0 / BLK), (128, E))
    bexp = jnp.sum((brow.astype(_f32) >= boffs).astype(_f32),
                   axis=1, keepdims=True) - 1.0
    brow1 = jax.lax.broadcasted_iota(jnp.int32, (128, 1), 0)
    bexp = jnp.where(brow1.astype(_f32) < used, bexp, -1.0)
    bexp_ref[...] = bexp.astype(jnp.int32)


def _routing(logits):
    return pl.pallas_call(
        _k4_body,
        out_shape=[
            jax.ShapeDtypeStruct((S, K), _f32),
            jax.ShapeDtypeStruct((K * S, 4), jnp.int32),
            jax.ShapeDtypeStruct((128, 1), jnp.int32),
        ],
    )(logits)


def _sc_mesh():
    return plsc.VectorSubcoreMesh(core_axis_name="core",
                                  subcore_axis_name="subcore")


def _dispatch_sc(xn2_q, idx2):
    @pl.kernel(out_type=jax.ShapeDtypeStruct((4 * PBUF, SCC), _f32),
               mesh=_sc_mesh(), scratch_types=[])
    def k(x_hbm, i_hbm, o_hbm):
        def body(x_vmem, i_vmem):
            pltpu.sync_copy(x_vmem, o_hbm.at[i_vmem.at[0]])
        pltpu.emit_pipeline(
            body,
            grid=(K, 4 * S // SCW),
            in_specs=[
                pl.BlockSpec((SCW, SCC), index_map=lambda kk, j: (j, 0)),
                pl.BlockSpec((1, SCW), index_map=lambda kk, j: (kk, j)),
            ],
            out_specs=[],
            core_axis_name=("core", "subcore"),
            dimension_semantics=(pltpu.PARALLEL, pltpu.PARALLEL),
        )(x_hbm, i_hbm)
    return k(xn2_q, idx2)


def _combine_sc(out_sorted_q, pos_flat):
    @pl.kernel(out_type=jax.ShapeDtypeStruct((4 * K * S, SCC), _f32),
               mesh=_sc_mesh(), scratch_types=[])
    def k(d_hbm, i_hbm, o_hbm):
        def body(i_vmem, o_vmem):
            pltpu.sync_copy(d_hbm.at[i_vmem.at[0]], o_vmem)
        pltpu.emit_pipeline(
            body,
            grid=(4 * K * S // SCW,),
            in_specs=[pl.BlockSpec((1, SCW), index_map=lambda j: (0, j))],
            out_specs=[pl.BlockSpec((SCW, SCC), index_map=lambda j: (j, 0))],
            core_axis_name=("core", "subcore"),
            dimension_semantics=(pltpu.PARALLEL,),
        )(i_hbm, o_hbm)
    return k(out_sorted_q, pos_flat)


def _k6_body(bexp_ref, x_ref, w1_ref, w2_ref, o_ref, w1b, w2b):
    i = pl.program_id(0)
    e = bexp_ref[i]
    eprev = jnp.where(i > 0, bexp_ref[jnp.maximum(i - 1, 0)], -2)

    @pl.when((e >= 0) & (e != eprev))
    def _():
        w1b[...] = w1_ref[0].astype(_bf16)
        w2b[...] = w2_ref[0].astype(_bf16)

    @pl.when(e >= 0)
    def _():
        x = x_ref[...].reshape(BLK, D).astype(_bf16)
        h1 = jnp.dot(x, w1b[...], preferred_element_type=_f32)
        g = h1[:, :FF]
        u = h1[:, FF:]
        act = (g * jax.lax.logistic(g) * u).astype(_bf16)
        out = jnp.dot(act, w2b[...], preferred_element_type=_f32)
        o_ref[...] = out.reshape(4 * BLK, SCC)


def _expert_ffn(bexp, x_sorted, w1, w2):
    def wmap(i, s):
        return (jnp.maximum(s[i], 0), 0, 0)

    grid_spec = pltpu.PrefetchScalarGridSpec(
        num_scalar_prefetch=1,
        grid=(NB,),
        in_specs=[
            pl.BlockSpec((4 * BLK, SCC), lambda i, s: (i, 0)),
            pl.BlockSpec((1, D, 2 * FF), wmap),
            pl.BlockSpec((1, FF, D), wmap),
        ],
        out_specs=pl.BlockSpec((4 * BLK, SCC), lambda i, s: (i, 0)),
        scratch_shapes=[
            pltpu.VMEM((D, 2 * FF), _bf16),
            pltpu.VMEM((FF, D), _bf16),
        ],
    )
    return pl.pallas_call(
        _k6_body,
        grid_spec=grid_spec,
        out_shape=jax.ShapeDtypeStruct((4 * PBUF, SCC), _f32),
    )(bexp, x_sorted, w1, w2)


def _k8_body(h2_ref, g0_ref, g1_ref, w_ref, o_ref):
    w0 = w_ref[:, 0:1]
    w1c = w_ref[:, 1:2]
    g0 = g0_ref[...].reshape(QB, D)
    g1 = g1_ref[...].reshape(QB, D)
    o_ref[...] = h2_ref[...] + w0 * g0 + w1c * g1


def _final(h2, g_all, wts):
    return pl.pallas_call(
        _k8_body,
        grid=(NQ,),
        in_specs=[
            pl.BlockSpec((QB, D), lambda i: (i, 0)),
            pl.BlockSpec((4 * QB, SCC), lambda i: (i, 0)),
            pl.BlockSpec((4 * QB, SCC), lambda i: (NQ + i, 0)),
            pl.BlockSpec((QB, K), lambda i: (i, 0)),
        ],
        out_specs=pl.BlockSpec((QB, D), lambda i: (i, 0)),
        out_shape=jax.ShapeDtypeStruct((S, D), _f32),
    )(h2, g_all, g_all, wts)


def kernel(hidden_states, ln1_gamma, ln2_gamma, wq, wk, wv, wo, gate_w, w1, w2):
    gw_pad = jnp.pad(gate_w, ((0, 0), (0, 128 - E))).astype(_bf16)
    g1 = ln1_gamma.reshape(1, D)
    g2 = ln2_gamma.reshape(1, D)

    q, k, v = _qkv(hidden_states, g1, wq.astype(_bf16), wk.astype(_bf16),
                   wv.astype(_bf16))
    attn = _attention(q, k, v)
    h2, xn2, logits = _post_attn(attn, wo.astype(_bf16), hidden_states, g2,
                                 gw_pad)
    wts, pos4, bexp = _routing(logits)

    pos_flat = pos4.reshape(1, 4 * K * S)
    idx2 = pos4.reshape(K, 4 * S)
    bexp_flat = bexp.reshape(-1)[:NB]

    x_sorted_q = _dispatch_sc(xn2, idx2)
    out_sorted_q = _expert_ffn(bexp_flat, x_sorted_q, w1, w2)
    g_q = _combine_sc(out_sorted_q, pos_flat)

    return _final(h2, g_q, wts)
